# Initial kernel scaffold; baseline (speedup 1.0000x reference)
import jax
import jax.numpy as jnp
from jax.experimental import pallas as pl


def kernel(x, edge_index, batch, W1, B1, W2, B2, W3, B3, g1, be1, g2, be2, g3, be3, LW1, Lb1, LW2, Lb2):
    raise NotImplementedError("write your pallas kernel here")


# speedup vs baseline: 5.9460x; 5.9460x over previous
import functools

import jax
import jax.numpy as jnp
from jax import lax
from jax.experimental import pallas as pl
from jax.experimental.pallas import tpu as pltpu
from jax.experimental.pallas import tpu_sc as plsc

N = 10000
D = 256
H = 256
E = 160000
G = 64
T = 12
EPS = 1e-5

NC = 2
NS = 16
HH = H // NC

K = 128
EW = 10240
NCHUNK = EW // K
EPAD = EW * NS
TRASH = N
NPAD = 10240
ZROWS = 64

RB = 2048
NB = NPAD // RB


def _sc_mesh():
    return plsc.VectorSubcoreMesh(
        core_axis_name="c", subcore_axis_name="s", num_cores=NC,
        num_subcores=NS)


def _make_deg():
    @functools.partial(
        pl.kernel,
        out_type=jax.ShapeDtypeStruct((NPAD, HH), jnp.float32),
        mesh=_sc_mesh(),
        scratch_types=[
            pltpu.VMEM((NCHUNK, K), jnp.int32),
            pltpu.VMEM((K,), jnp.int32),
            pltpu.VMEM((K,), jnp.int32),
            pltpu.VMEM((K, HH), jnp.float32),
            pltpu.VMEM((K, HH), jnp.float32),
            pltpu.VMEM_SHARED((NPAD, HH), jnp.float32),
            pltpu.SemaphoreType.DMA,
        ],
    )
    def deg_kernel(dst_ref, ones_ref, z_ref, out_ref,
                   dst_v, dst_cur, idx_cur, ones_v, ebuf, accd, dsem):
        c = lax.axis_index("c")
        s = lax.axis_index("s")

        pltpu.sync_copy(z_ref, ebuf)
        zbase = s * (NPAD // NS)

        def zero_step(j, _):
            pltpu.sync_copy(ebuf, accd.at[pl.ds(zbase + j * K, K)])
            return 0

        lax.fori_loop(0, NPAD // NS // K, zero_step, 0)
        pltpu.sync_copy(dst_ref.at[s], dst_v)
        pltpu.sync_copy(ones_ref, ones_v)
        plsc.subcore_barrier()

        def step(j, _):
            def stage(l, _):
                dst_cur[pl.ds(l * 16, 16)] = dst_v[j, pl.ds(l * 16, 16)]
                return 0

            lax.fori_loop(0, K // 16, stage, 0)
            pltpu.sync_copy(ones_v, accd.at[dst_cur], add=True)
            return 0

        lax.fori_loop(0, NCHUNK, step, 0)
        plsc.subcore_barrier()

        @pl.when(c == 0)
        def _():
            def out_step(j, _):
                def fill(l, _):
                    idx_cur[pl.ds(l * 16, 16)] = (
                        zbase + j * K + l * 16 + lax.iota(jnp.int32, 16))
                    return 0

                lax.fori_loop(0, K // 16, fill, 0)
                pltpu.async_copy(accd.at[idx_cur], ebuf, dsem).wait()
                pltpu.sync_copy(ebuf, out_ref.at[pl.ds(zbase + j * K, K)])
                return 0

            lax.fori_loop(0, NPAD // NS // K, out_step, 0)

    return deg_kernel


_CACHE = {}


def _deg_kernel():
    if "deg" not in _CACHE:
        _CACHE["deg"] = _make_deg()
    return _CACHE["deg"]


def _make_agg():
    @functools.partial(
        pl.kernel,
        out_type=jax.ShapeDtypeStruct((2 * NPAD, HH), jnp.float32),
        mesh=_sc_mesh(),
        scratch_types=[
            pltpu.VMEM((NCHUNK, K), jnp.int32),
            pltpu.VMEM((NCHUNK, K), jnp.int32),
            pltpu.VMEM((K,), jnp.int32),
            pltpu.VMEM((K,), jnp.int32),
            pltpu.VMEM((K,), jnp.int32),
            pltpu.VMEM((K, HH), jnp.float32),
            pltpu.VMEM_SHARED((NPAD, HH), jnp.float32),
            pltpu.SemaphoreType.DMA,
        ],
    )
    def agg_kernel(p_ref, src_ref, dst_ref, z_ref, out_ref,
                   src_v, dst_v, src_cur, dst_cur, idx_cur, rows_v, acc,
                   sem):
        c = lax.axis_index("c")
        s = lax.axis_index("s")

        pltpu.sync_copy(z_ref, rows_v)
        zbase = s * (NPAD // NS)

        def zero_step(j, _):
            pltpu.sync_copy(rows_v, acc.at[pl.ds(zbase + j * K, K)])
            return 0

        lax.fori_loop(0, NPAD // NS // K, zero_step, 0)

        pltpu.sync_copy(src_ref.at[c * NS + s], src_v)
        pltpu.sync_copy(dst_ref.at[s], dst_v)
        plsc.subcore_barrier()

        def step(j, _):
            def stage(l, _):
                src_cur[pl.ds(l * 16, 16)] = src_v[j, pl.ds(l * 16, 16)]
                dst_cur[pl.ds(l * 16, 16)] = dst_v[j, pl.ds(l * 16, 16)]
                return 0

            lax.fori_loop(0, K // 16, stage, 0)
            pltpu.async_copy(p_ref.at[src_cur], rows_v, sem).wait()
            pltpu.sync_copy(rows_v, acc.at[dst_cur], add=True)
            return 0

        lax.fori_loop(0, NCHUNK, step, 0)
        plsc.subcore_barrier()

        def out_step(j, _):
            def fill(l, _):
                idx_cur[pl.ds(l * 16, 16)] = (
                    zbase + j * K + l * 16 + lax.iota(jnp.int32, 16))
                return 0

            lax.fori_loop(0, K // 16, fill, 0)
            pltpu.async_copy(acc.at[idx_cur], rows_v, sem).wait()
            pltpu.sync_copy(
                rows_v, out_ref.at[pl.ds(c * NPAD + zbase + j * K, K)])
            return 0

        lax.fori_loop(0, NPAD // NS // K, out_step, 0)

    return agg_kernel


def _agg_kernel():
    if "agg" not in _CACHE:
        _CACHE["agg"] = _make_agg()
    return _CACHE["agg"]


def _dinv_of(deg_blk):
    return lax.rsqrt(deg_blk + 1.0)


def _a1_kernel(x_ref, w_ref, deg_ref, p_ref):
    dinv = _dinv_of(deg_ref[...])
    h = jnp.dot(x_ref[...], w_ref[...], preferred_element_type=jnp.float32)
    p_ref[...] = h * dinv


def _a1_call(x, w, deg3):
    return pl.pallas_call(
        _a1_kernel,
        grid=(NB, NC),
        in_specs=[
            pl.BlockSpec((RB, D), lambda i, j: (i, 0)),
            pl.BlockSpec((D, HH), lambda i, j: (0, j)),
            pl.BlockSpec((RB, 1), lambda i, j: (i, 0)),
        ],
        out_specs=pl.BlockSpec((RB, HH), lambda i, j: (j * NB + i, 0)),
        out_shape=jax.ShapeDtypeStruct((2 * NPAD, HH), jnp.float32),
    )(x, w, deg3)


def _a_kernel(t_ref, w_ref, deg_ref, sums_ref, g_ref, be_ref, p_ref):
    m = sums_ref[0:1, :] * (1.0 / N)
    v = sums_ref[1:2, :] * (1.0 / N) - m * m
    a = g_ref[...] * lax.rsqrt(v + EPS)
    cc = be_ref[...] - m * a
    xin = jnp.maximum(a * t_ref[...] + cc, 0.0)
    dinv = _dinv_of(deg_ref[...])
    h = jnp.dot(xin, w_ref[...], preferred_element_type=jnp.float32)
    p_ref[...] = h * dinv


def _a_call(t, w, deg3, sums, g, be):
    return pl.pallas_call(
        _a_kernel,
        grid=(NB, NC),
        in_specs=[
            pl.BlockSpec((RB, H), lambda i, j: (i, 0)),
            pl.BlockSpec((H, HH), lambda i, j: (0, j)),
            pl.BlockSpec((RB, 1), lambda i, j: (i, 0)),
            pl.BlockSpec((8, H), lambda i, j: (0, 0)),
            pl.BlockSpec((1, H), lambda i, j: (0, 0)),
            pl.BlockSpec((1, H), lambda i, j: (0, 0)),
        ],
        out_specs=pl.BlockSpec((RB, HH), lambda i, j: (j * NB + i, 0)),
        out_shape=jax.ShapeDtypeStruct((2 * NPAD, HH), jnp.float32),
    )(t, w, deg3, sums, g, be)


def _c_kernel(s0_ref, s1_ref, p0_ref, p1_ref, deg_ref, b_ref,
              t_ref, sums_ref):
    i = pl.program_id(0)
    dinv = _dinv_of(deg_ref[...])
    t0 = dinv * (s0_ref[...] + p0_ref[...]) + b_ref[:, :HH]
    t1 = dinv * (s1_ref[...] + p1_ref[...]) + b_ref[:, HH:]
    t_ref[:, :HH] = t0
    t_ref[:, HH:] = t1

    @pl.when(i == 0)
    def _():
        sums_ref[...] = jnp.zeros((8, H), jnp.float32)

    rid = lax.broadcasted_iota(jnp.int32, (RB, 1), 0) + i * RB
    msk = (rid < N).astype(jnp.float32)
    t0m = t0 * msk
    t1m = t1 * msk
    sums_ref[0:1, :HH] += jnp.sum(t0m, axis=0, keepdims=True)
    sums_ref[0:1, HH:] += jnp.sum(t1m, axis=0, keepdims=True)
    sums_ref[1:2, :HH] += jnp.sum(t0m * t0m, axis=0, keepdims=True)
    sums_ref[1:2, HH:] += jnp.sum(t1m * t1m, axis=0, keepdims=True)


def _c_call(S, p, deg3, b):
    return pl.pallas_call(
        _c_kernel,
        grid=(NB,),
        in_specs=[
            pl.BlockSpec((RB, HH), lambda i: (i, 0)),
            pl.BlockSpec((RB, HH), lambda i: (NB + i, 0)),
            pl.BlockSpec((RB, HH), lambda i: (i, 0)),
            pl.BlockSpec((RB, HH), lambda i: (NB + i, 0)),
            pl.BlockSpec((RB, 1), lambda i: (i, 0)),
            pl.BlockSpec((1, H), lambda i: (0, 0)),
        ],
        out_specs=[
            pl.BlockSpec((RB, H), lambda i: (i, 0)),
            pl.BlockSpec((8, H), lambda i: (0, 0)),
        ],
        out_shape=[
            jax.ShapeDtypeStruct((NPAD, H), jnp.float32),
            jax.ShapeDtypeStruct((8, H), jnp.float32),
        ],
    )(S, S, p, p, deg3, b)


def _pool_kernel(t_ref, sums_ref, g_ref, be_ref, batch_ref,
                 lw1a_ref, lw1b_ref, lb1_ref, lw2_ref, lb2_ref,
                 y_ref, s_scr, cnt_scr):
    i = pl.program_id(0)

    @pl.when(i == 0)
    def _():
        s_scr[...] = jnp.zeros((G, H), jnp.float32)
        cnt_scr[...] = jnp.zeros((G, 128), jnp.float32)

    m = sums_ref[0:1, :] * (1.0 / N)
    v = sums_ref[1:2, :] * (1.0 / N) - m * m
    a = g_ref[...] * lax.rsqrt(v + EPS)
    cc = be_ref[...] - m * a
    x3 = jnp.maximum(a * t_ref[...] + cc, 0.0)

    gids = lax.broadcasted_iota(jnp.int32, (RB, G), 1)
    onehot = (batch_ref[...] == gids).astype(jnp.float32)
    s_scr[...] += lax.dot_general(
        onehot, x3, (((0,), (0,)), ((), ())),
        preferred_element_type=jnp.float32)
    cnt_scr[:, 0:1] += lax.dot_general(
        onehot, jnp.ones((RB, 1), jnp.float32), (((0,), (0,)), ((), ())),
        preferred_element_type=jnp.float32)

    @pl.when(i == NB - 1)
    def _():
        sg = s_scr[...]
        cnt = jnp.maximum(cnt_scr[:, 0:1], 1.0)
        mean = sg / cnt
        hid = jnp.maximum(
            jnp.dot(mean, lw1a_ref[...], preferred_element_type=jnp.float32)
            + jnp.dot(sg, lw1b_ref[...], preferred_element_type=jnp.float32)
            + lb1_ref[...], 0.0)
        y_ref[...] = (
            jnp.dot(hid, lw2_ref[...], preferred_element_type=jnp.float32)
            + lb2_ref[...])


def _pool_call(t3, sums3, g3, be3, batch_col, lw1a, lw1b, lb1, lw2p, lb2p):
    return pl.pallas_call(
        _pool_kernel,
        grid=(NB,),
        in_specs=[
            pl.BlockSpec((RB, H), lambda i: (i, 0)),
            pl.BlockSpec((8, H), lambda i: (0, 0)),
            pl.BlockSpec((1, H), lambda i: (0, 0)),
            pl.BlockSpec((1, H), lambda i: (0, 0)),
            pl.BlockSpec((RB, 1), lambda i: (i, 0)),
            pl.BlockSpec((H, H), lambda i: (0, 0)),
            pl.BlockSpec((H, H), lambda i: (0, 0)),
            pl.BlockSpec((1, H), lambda i: (0, 0)),
            pl.BlockSpec((H, 128), lambda i: (0, 0)),
            pl.BlockSpec((1, 128), lambda i: (0, 0)),
        ],
        out_specs=pl.BlockSpec((G, 128), lambda i: (0, 0)),
        out_shape=jax.ShapeDtypeStruct((G, 128), jnp.float32),
        scratch_shapes=[
            pltpu.VMEM((G, H), jnp.float32),
            pltpu.VMEM((G, 128), jnp.float32),
        ],
    )(t3, sums3, g3, be3, batch_col, lw1a, lw1b, lb1, lw2p, lb2p)


def kernel(x, edge_index, batch, W1, B1, W2, B2, W3, B3,
           g1, be1, g2, be2, g3, be3, LW1, Lb1, LW2, Lb2):
    src = edge_index[0]
    dst = edge_index[1]
    pad = EPAD - E
    src_p = jnp.concatenate([src, jnp.zeros((pad,), jnp.int32)])
    dst_p = jnp.concatenate([dst, jnp.full((pad,), TRASH, jnp.int32)])
    src_r = src_p.reshape(NS, NCHUNK, K)
    src_b = jnp.concatenate([src_r, src_r + NPAD], axis=0)
    dst_r = dst_p.reshape(NS, NCHUNK, K)
    zrows = jnp.zeros((K, HH), jnp.float32)
    ones_k = jnp.ones((K, HH), jnp.float32)
    z16 = zrows

    degp = _deg_kernel()(dst_r, ones_k, z16)
    deg3 = degp[:, 0:1]

    xpad = jnp.pad(x, ((0, NPAD - N), (0, 0)))
    batch_col = jnp.pad(batch, (0, NPAD - N),
                        constant_values=G).reshape(NPAD, 1)

    row = lambda a: a.reshape(1, -1)

    p1 = _a1_call(xpad, W1, deg3)
    S1 = _agg_kernel()(p1, src_b, dst_r, zrows)
    t1, sums1 = _c_call(S1, p1, deg3, row(B1))

    p2 = _a_call(t1, W2, deg3, sums1, row(g1), row(be1))
    S2 = _agg_kernel()(p2, src_b, dst_r, zrows)
    t2, sums2 = _c_call(S2, p2, deg3, row(B2))

    p3 = _a_call(t2, W3, deg3, sums2, row(g2), row(be2))
    S3 = _agg_kernel()(p3, src_b, dst_r, zrows)
    t3, sums3 = _c_call(S3, p3, deg3, row(B3))

    lw2p = jnp.pad(LW2, ((0, 0), (0, 128 - T)))
    lb2p = jnp.pad(Lb2, (0, 128 - T)).reshape(1, 128)
    y = _pool_call(t3, sums3, row(g3), row(be3), batch_col,
                   LW1[:H], LW1[H:], row(Lb1), lw2p, lb2p)
    return y[:, :T]
